# Initial kernel scaffold; baseline (speedup 1.0000x reference)
import jax
import jax.numpy as jnp
from jax.experimental import pallas as pl


def kernel(x, edge_index, edge_type, basis0, comp0, root0, bias0, basis1, comp1, root1, bias1, basis2, comp2, root2, bias2):
    raise NotImplementedError("write your pallas kernel here")


# speedup vs baseline: 9.2320x; 9.2320x over previous
import functools

import jax
import jax.numpy as jnp
from jax import lax
from jax.experimental import pallas as pl
from jax.experimental.pallas import tpu as pltpu
from jax.experimental.pallas import tpu_sc as plsc

N = 10000
E = 320000
R = 6
NSEG = N * R
NC, NS, LANES = 2, 16, 16

W = 32
BLK = 128
NBLKS = 160
EPAD = NS * NBLKS * BLK
GRP = 4
NGRP = NBLKS // GRP
ACC_ROWS = 60288
NPAD = ACC_ROWS // R
TROWS = ACC_ROWS // NS
SEGMASK = 0x1FFFF

BM = 1000


def _sc_aggregate_body(C, h2, pk2, out, acc,
                       pk0, sg0, pk1, sg1, rows0, rows1,
                       gs0, gs1, ss0, ss1, is0, is1):
    CPC = C // NC
    core = lax.axis_index("c")
    sub = lax.axis_index("s")

    def chunk_body(ci, _):
        c = core * CPC + ci
        def zfill(i, _):
            for j in range(W // LANES):
                rows0[i, pl.ds(j * LANES, LANES)] = jnp.zeros(
                    (LANES,), jnp.float32)
                rows1[i, pl.ds(j * LANES, LANES)] = jnp.zeros(
                    (LANES,), jnp.float32)
            return 0
        lax.fori_loop(0, BLK, zfill, 0)
        zbase = sub * TROWS
        zsrc = (rows0, rows1)
        zsem = (gs0, gs1)
        for z in range(TROWS // BLK):
            pltpu.async_copy(zsrc[z % 2], acc.at[pl.ds(zbase + z * BLK, BLK)],
                             zsem[z % 2])
        rem = TROWS % BLK
        if rem:
            pltpu.async_copy(rows0.at[pl.ds(0, rem)],
                             acc.at[pl.ds(zbase + (TROWS // BLK) * BLK, rem)],
                             gs0)
        for z in range(TROWS // BLK):
            pltpu.make_async_copy(
                zsrc[z % 2], acc.at[pl.ds(zbase + z * BLK, BLK)],
                zsem[z % 2]).wait()
        if rem:
            pltpu.make_async_copy(
                rows0.at[pl.ds(0, rem)],
                acc.at[pl.ds(zbase + (TROWS // BLK) * BLK, rem)], gs0).wait()
        plsc.subcore_barrier()

        rows = (rows0, rows1)
        gsem = (gs0, gs1)
        ssem = (ss0, ss1)

        def issue_group(g, pkb, sem):
            pltpu.async_copy(pk2.at[pl.ds(sub * NBLKS + g * GRP, GRP)],
                             pkb, sem)

        def unpack_group(g, pkb, sgb, sem):
            pltpu.make_async_copy(
                pk2.at[pl.ds(sub * NBLKS + g * GRP, GRP)], pkb, sem).wait()
            for i in range(GRP):
                for j in range(128 // LANES):
                    v = pkb[i, pl.ds(j * LANES, LANES)]
                    sgb[i, pl.ds(j * LANES, LANES)] = v & SEGMASK
                    pkb[i, pl.ds(j * LANES, LANES)] = (
                        lax.shift_right_logical(v, 17) * C + c)

        def gather(pkb, i, x):
            pltpu.async_copy(h2.at[pkb.at[i]], rows[x], gsem[x])

        def wait_gather(pkb, i, x):
            pltpu.make_async_copy(h2.at[pkb.at[i]], rows[x], gsem[x]).wait()

        def scatter(sgb, i, x):
            pltpu.async_copy(rows[x], acc.at[sgb.at[i]], ssem[x], add=True)

        def wait_scatter(sgb, i, x):
            pltpu.make_async_copy(rows[x], acc.at[sgb.at[i]], ssem[x]).wait()

        def steady(pk_b, i_b, sg_w, i_w, pk_r, sg_r, i_r, x):
            wait_scatter(sg_w, i_w, x)
            gather(pk_b, i_b, x)
            wait_gather(pk_r, i_r, 1 - x)
            scatter(sg_r, i_r, 1 - x)

        issue_group(0, pk0, is0)
        issue_group(1, pk1, is1)
        unpack_group(0, pk0, sg0, is0)
        gather(pk0, 0, 0)
        gather(pk0, 1, 1)
        wait_gather(pk0, 0, 0)
        scatter(sg0, 0, 0)
        wait_scatter(sg0, 0, 0)
        gather(pk0, 2, 0)
        wait_gather(pk0, 1, 1)
        scatter(sg0, 1, 1)
        steady(pk0, 3, sg0, 1, pk0, sg0, 2, 1)
        unpack_group(1, pk1, sg1, is1)
        steady(pk1, 0, sg0, 2, pk0, sg0, 3, 0)
        steady(pk1, 1, sg0, 3, pk1, sg1, 0, 1)
        issue_group(2, pk0, is0)
        steady(pk1, 2, sg1, 0, pk1, sg1, 1, 0)
        steady(pk1, 3, sg1, 1, pk1, sg1, 2, 1)
        unpack_group(2, pk0, sg0, is0)

        def dsuper(ds_, _):
            steady(pk0, 0, sg1, 2, pk1, sg1, 3, 0)
            steady(pk0, 1, sg1, 3, pk0, sg0, 0, 1)
            issue_group(2 * ds_ + 1, pk1, is1)
            steady(pk0, 2, sg0, 0, pk0, sg0, 1, 0)
            steady(pk0, 3, sg0, 1, pk0, sg0, 2, 1)
            unpack_group(2 * ds_ + 1, pk1, sg1, is1)
            steady(pk1, 0, sg0, 2, pk0, sg0, 3, 0)
            steady(pk1, 1, sg0, 3, pk1, sg1, 0, 1)

            @pl.when(2 * ds_ + 2 < NGRP)
            def _():
                issue_group(2 * ds_ + 2, pk0, is0)
            steady(pk1, 2, sg1, 0, pk1, sg1, 1, 0)
            steady(pk1, 3, sg1, 1, pk1, sg1, 2, 1)

            @pl.when(2 * ds_ + 2 < NGRP)
            def _():
                unpack_group(2 * ds_ + 2, pk0, sg0, is0)
            return 0
        lax.fori_loop(1, NGRP // 2, dsuper, 0)

        wait_gather(pk1, GRP - 1, 1)
        scatter(sg1, GRP - 1, 1)
        wait_scatter(sg1, GRP - 2, 0)
        wait_scatter(sg1, GRP - 1, 1)
        plsc.subcore_barrier()

        fb = sub * TROWS
        pltpu.sync_copy(acc.at[pl.ds(fb, TROWS)], out.at[c, pl.ds(fb, TROWS)])
        plsc.subcore_barrier()
        return 0
    lax.fori_loop(0, CPC, chunk_body, 0)


def _sc_aggregate(h2, pk2, C):
    mesh = plsc.VectorSubcoreMesh(core_axis_name="c", subcore_axis_name="s")
    return pl.kernel(
        functools.partial(_sc_aggregate_body, C),
        out_type=jax.ShapeDtypeStruct((C, ACC_ROWS, W), jnp.float32),
        mesh=mesh,
        scratch_types=[
            pltpu.VMEM_SHARED((ACC_ROWS, W), jnp.float32),
            pltpu.VMEM((GRP, 128), jnp.int32),
            pltpu.VMEM((GRP, 128), jnp.int32),
            pltpu.VMEM((GRP, 128), jnp.int32),
            pltpu.VMEM((GRP, 128), jnp.int32),
            pltpu.VMEM((BLK, W), jnp.float32),
            pltpu.VMEM((BLK, W), jnp.float32),
            pltpu.SemaphoreType.DMA,
            pltpu.SemaphoreType.DMA,
            pltpu.SemaphoreType.DMA,
            pltpu.SemaphoreType.DMA,
            pltpu.SemaphoreType.DMA,
            pltpu.SemaphoreType.DMA,
        ],
        compiler_params=pltpu.CompilerParams(use_tc_tiling_on_sc=False),
        name="sc_rgcn_aggregate",
    )(h2, pk2)


def _sc_counts_body(pk2, out, acc, pkb, sgb, buf):
    core = lax.axis_index("c")
    sub = lax.axis_index("s")

    @pl.when(core == 0)
    def _():
        def fill(val):
            def f(i, _):
                for j in range(W // LANES):
                    buf[i, pl.ds(j * LANES, LANES)] = jnp.full(
                        (LANES,), val, jnp.float32)
                return 0
            lax.fori_loop(0, BLK, f, 0)

        fill(0.0)
        zbase = sub * TROWS
        for z in range(TROWS // BLK):
            pltpu.sync_copy(buf, acc.at[pl.ds(zbase + z * BLK, BLK)])
        rem = TROWS % BLK
        if rem:
            pltpu.sync_copy(buf.at[pl.ds(0, rem)],
                            acc.at[pl.ds(zbase + (TROWS // BLK) * BLK, rem)])
        fill(1.0)
        plsc.subcore_barrier()

        def grp_body(g, _):
            pltpu.sync_copy(pk2.at[pl.ds(sub * NBLKS + g * GRP, GRP)], pkb)
            for i in range(GRP):
                for j in range(128 // LANES):
                    v = pkb[i, pl.ds(j * LANES, LANES)]
                    sgb[i, pl.ds(j * LANES, LANES)] = v & SEGMASK
            for i in range(GRP):
                pltpu.sync_copy(buf, acc.at[sgb.at[i]], add=True)
            return 0
        lax.fori_loop(0, NGRP, grp_body, 0)
        plsc.subcore_barrier()

        fb = sub * TROWS
        pltpu.sync_copy(acc.at[pl.ds(fb, TROWS)], out.at[pl.ds(fb, TROWS)])

    return


def _sc_counts(pk2):
    mesh = plsc.VectorSubcoreMesh(core_axis_name="c", subcore_axis_name="s")
    return pl.kernel(
        _sc_counts_body,
        out_type=jax.ShapeDtypeStruct((ACC_ROWS, W), jnp.float32),
        mesh=mesh,
        scratch_types=[
            pltpu.VMEM_SHARED((ACC_ROWS, W), jnp.float32),
            pltpu.VMEM((GRP, 128), jnp.int32),
            pltpu.VMEM((GRP, 128), jnp.int32),
            pltpu.VMEM((BLK, W), jnp.float32),
        ],
        compiler_params=pltpu.CompilerParams(use_tc_tiling_on_sc=False),
        name="sc_rgcn_counts",
    )(pk2)


def _tc_wrel_body(basis_ref, comp_ref, out_ref):
    r = pl.program_id(1)
    acc = comp_ref[r, 0] * basis_ref[0]
    acc = acc + comp_ref[r, 1] * basis_ref[1]
    acc = acc + comp_ref[r, 2] * basis_ref[2]
    out_ref[0] = acc


def _tc_wrel(basis, comp, C, dout):
    return pl.pallas_call(
        _tc_wrel_body,
        grid=(C, R),
        in_specs=[
            pl.BlockSpec((3, W, dout), lambda c, r: (0, c, 0)),
            pl.BlockSpec(memory_space=pltpu.SMEM),
        ],
        out_specs=pl.BlockSpec((1, W, dout), lambda c, r: (c, r, 0)),
        out_shape=jax.ShapeDtypeStruct((C, R * W, dout), jnp.float32),
        name="tc_wrel",
    )(basis, comp)


def _tc_inv_body(cnt_ref, out_ref):
    out_ref[...] = 1.0 / jnp.maximum(cnt_ref[...], 1.0)


def _tc_inv(cntx):
    return pl.pallas_call(
        _tc_inv_body,
        grid=(8,),
        in_specs=[pl.BlockSpec((NPAD // 8, R * W), lambda m: (m, 0))],
        out_specs=pl.BlockSpec((NPAD // 8, R * W), lambda m: (m, 0)),
        out_shape=jax.ShapeDtypeStruct((NPAD, R * W), jnp.float32),
        name="tc_inv_counts",
    )(cntx)


def _tc_root_body(h_ref, root_ref, bias_ref, out_ref):
    out_ref[...] = jnp.dot(h_ref[...], root_ref[...],
                           preferred_element_type=jnp.float32) + bias_ref[...]


def _tc_root(h, root, bias):
    din, dout = root.shape
    return pl.pallas_call(
        _tc_root_body,
        grid=(N // BM,),
        in_specs=[
            pl.BlockSpec((BM, din), lambda m: (m, 0)),
            pl.BlockSpec((din, dout), lambda m: (0, 0)),
            pl.BlockSpec((1, dout), lambda m: (0, 0)),
        ],
        out_specs=pl.BlockSpec((BM, dout), lambda m: (m, 0)),
        out_shape=jax.ShapeDtypeStruct((N, dout), jnp.float32),
        name="tc_root",
    )(h, root, bias)


def _tc_layer_body(C2, act, part0_ref, wrel_ref, sums_ref, inv_ref,
                   out_ref, acc_ref):
    m = pl.program_id(0)
    k = pl.program_id(1)
    invb = inv_ref[...]
    part = (jnp.dot(sums_ref[0] * invb, wrel_ref[0],
                    preferred_element_type=jnp.float32)
            + jnp.dot(sums_ref[1] * invb, wrel_ref[1],
                      preferred_element_type=jnp.float32))

    @pl.when(k == 0)
    def _():
        acc_ref[...] = part + part0_ref[...]

    @pl.when(k > 0)
    def _():
        acc_ref[...] = acc_ref[...] + part

    @pl.when(k == C2 - 1)
    def _():
        val = acc_ref[...]
        if act == "relu":
            out_ref[...] = jnp.maximum(val, 0.0)
        else:
            val = jnp.tanh(val)
            bmax = jnp.max(val, axis=0, keepdims=True)

            @pl.when(m == 0)
            def _():
                out_ref[...] = bmax

            @pl.when(m > 0)
            def _():
                out_ref[...] = jnp.maximum(out_ref[...], bmax)


def _tc_layer(part0, din, wrel, sums, inv, act):
    dout = part0.shape[1]
    C2 = din // W // 2
    if act == "relu":
        out_shape = jax.ShapeDtypeStruct((N, dout), jnp.float32)
        out_spec = pl.BlockSpec((BM, dout), lambda m, k: (m, 0))
    else:
        out_shape = jax.ShapeDtypeStruct((1, dout), jnp.float32)
        out_spec = pl.BlockSpec((1, dout), lambda m, k: (0, 0))
    return pl.pallas_call(
        functools.partial(_tc_layer_body, C2, act),
        grid=(N // BM, C2),
        in_specs=[
            pl.BlockSpec((BM, dout), lambda m, k: (m, 0)),
            pl.BlockSpec((2, R * W, dout), lambda m, k: (k, 0, 0)),
            pl.BlockSpec((2, BM, R * W), lambda m, k: (k, m, 0)),
            pl.BlockSpec((BM, R * W), lambda m, k: (m, 0)),
        ],
        out_specs=out_spec,
        out_shape=out_shape,
        scratch_shapes=[pltpu.VMEM((BM, dout), jnp.float32)],
        name="tc_rgcn_layer",
    )(part0, wrel, sums, inv)


def kernel(x, edge_index, edge_type, basis0, comp0, root0, bias0,
           basis1, comp1, root1, bias1, basis2, comp2, root2, bias2):
    src = edge_index[0]
    dst = edge_index[1]
    seg = dst * R + edge_type
    pk = jnp.bitwise_or(jnp.left_shift(src, 17), seg)
    pad = EPAD - E
    pk2 = jnp.concatenate(
        [pk, jnp.full((pad,), NSEG, jnp.int32)]).reshape(EPAD // 128, 128)

    cntx = _sc_counts(pk2).reshape(NPAD, R * W)
    inv = _tc_inv(cntx)

    h = x
    layers = [(basis0, comp0, root0, bias0, "relu"),
              (basis1, comp1, root1, bias1, "relu"),
              (basis2, comp2, root2, bias2, "tanh")]
    for basis, comp, root, bias, act in layers:
        din, dout = root.shape
        C = din // W
        wrel = _tc_wrel(basis, comp, C, dout)
        sums = _sc_aggregate(h.reshape(N * C, W), pk2, C)
        part0 = _tc_root(h, root, bias.reshape(1, dout))
        h = _tc_layer(part0, din, wrel, sums.reshape(C, NPAD, R * W), inv,
                      act)
    return h


# speedup vs baseline: 9.4634x; 1.0251x over previous
import functools

import jax
import jax.numpy as jnp
from jax import lax
from jax.experimental import pallas as pl
from jax.experimental.pallas import tpu as pltpu
from jax.experimental.pallas import tpu_sc as plsc

N = 10000
E = 320000
R = 6
NSEG = N * R
NC, NS, LANES = 2, 16, 16

W = 32
BLK = 128
NBLKS = 160
EPAD = NS * NBLKS * BLK
GRP = 4
NGRP = NBLKS // GRP
ACC_ROWS = 60288
NPAD = ACC_ROWS // R
TROWS = ACC_ROWS // NS
SEGMASK = 0x1FFFF

BM = 1000


def _sc_aggregate_body(C, h2, pk2, out, acc,
                       pk0, sg0, pk1, sg1, rows0, rows1,
                       gs0, gs1, ss0, ss1, is0, is1):
    CPC = C // NC
    core = lax.axis_index("c")
    sub = lax.axis_index("s")

    zbase = sub * TROWS
    NSLAB = TROWS // BLK
    ZREM = TROWS % BLK
    zsrc = (rows0, rows1)
    zsem = (gs0, gs1)

    def zfill(i, _):
        for j in range(W // LANES):
            rows0[i, pl.ds(j * LANES, LANES)] = jnp.zeros(
                (LANES,), jnp.float32)
            rows1[i, pl.ds(j * LANES, LANES)] = jnp.zeros(
                (LANES,), jnp.float32)
        return 0

    def zero_slab(z):
        pltpu.async_copy(zsrc[z % 2], acc.at[pl.ds(zbase + z * BLK, BLK)],
                         zsem[z % 2])

    def wait_zero_slab(z):
        pltpu.make_async_copy(zsrc[z % 2],
                              acc.at[pl.ds(zbase + z * BLK, BLK)],
                              zsem[z % 2]).wait()

    def zero_rem():
        pltpu.async_copy(rows0.at[pl.ds(0, ZREM)],
                         acc.at[pl.ds(zbase + NSLAB * BLK, ZREM)], gs0)

    def wait_zero_rem():
        pltpu.make_async_copy(rows0.at[pl.ds(0, ZREM)],
                              acc.at[pl.ds(zbase + NSLAB * BLK, ZREM)],
                              gs0).wait()

    lax.fori_loop(0, BLK, zfill, 0)
    for z in range(NSLAB):
        zero_slab(z)
    zero_rem()
    for z in range(NSLAB):
        wait_zero_slab(z)
    wait_zero_rem()
    plsc.subcore_barrier()

    def chunk_body(ci, _):
        c = core * CPC + ci

        rows = (rows0, rows1)
        gsem = (gs0, gs1)
        ssem = (ss0, ss1)

        def issue_group(g, pkb, sem):
            pltpu.async_copy(pk2.at[pl.ds(sub * NBLKS + g * GRP, GRP)],
                             pkb, sem)

        def unpack_group(g, pkb, sgb, sem):
            pltpu.make_async_copy(
                pk2.at[pl.ds(sub * NBLKS + g * GRP, GRP)], pkb, sem).wait()
            for i in range(GRP):
                for j in range(128 // LANES):
                    v = pkb[i, pl.ds(j * LANES, LANES)]
                    sgb[i, pl.ds(j * LANES, LANES)] = v & SEGMASK
                    pkb[i, pl.ds(j * LANES, LANES)] = (
                        lax.shift_right_logical(v, 17) * C + c)

        def gather(pkb, i, x):
            pltpu.async_copy(h2.at[pkb.at[i]], rows[x], gsem[x])

        def wait_gather(pkb, i, x):
            pltpu.make_async_copy(h2.at[pkb.at[i]], rows[x], gsem[x]).wait()

        def scatter(sgb, i, x):
            pltpu.async_copy(rows[x], acc.at[sgb.at[i]], ssem[x], add=True)

        def wait_scatter(sgb, i, x):
            pltpu.make_async_copy(rows[x], acc.at[sgb.at[i]], ssem[x]).wait()

        def steady(pk_b, i_b, sg_w, i_w, pk_r, sg_r, i_r, x):
            wait_scatter(sg_w, i_w, x)
            gather(pk_b, i_b, x)
            wait_gather(pk_r, i_r, 1 - x)
            scatter(sg_r, i_r, 1 - x)

        issue_group(0, pk0, is0)
        issue_group(1, pk1, is1)
        unpack_group(0, pk0, sg0, is0)
        gather(pk0, 0, 0)
        gather(pk0, 1, 1)
        wait_gather(pk0, 0, 0)
        scatter(sg0, 0, 0)
        wait_scatter(sg0, 0, 0)
        gather(pk0, 2, 0)
        wait_gather(pk0, 1, 1)
        scatter(sg0, 1, 1)
        steady(pk0, 3, sg0, 1, pk0, sg0, 2, 1)
        unpack_group(1, pk1, sg1, is1)
        steady(pk1, 0, sg0, 2, pk0, sg0, 3, 0)
        steady(pk1, 1, sg0, 3, pk1, sg1, 0, 1)
        issue_group(2, pk0, is0)
        steady(pk1, 2, sg1, 0, pk1, sg1, 1, 0)
        steady(pk1, 3, sg1, 1, pk1, sg1, 2, 1)
        unpack_group(2, pk0, sg0, is0)

        def dsuper(ds_, _):
            steady(pk0, 0, sg1, 2, pk1, sg1, 3, 0)
            steady(pk0, 1, sg1, 3, pk0, sg0, 0, 1)
            issue_group(2 * ds_ + 1, pk1, is1)
            steady(pk0, 2, sg0, 0, pk0, sg0, 1, 0)
            steady(pk0, 3, sg0, 1, pk0, sg0, 2, 1)
            unpack_group(2 * ds_ + 1, pk1, sg1, is1)
            steady(pk1, 0, sg0, 2, pk0, sg0, 3, 0)
            steady(pk1, 1, sg0, 3, pk1, sg1, 0, 1)

            @pl.when(2 * ds_ + 2 < NGRP)
            def _():
                issue_group(2 * ds_ + 2, pk0, is0)
            steady(pk1, 2, sg1, 0, pk1, sg1, 1, 0)
            steady(pk1, 3, sg1, 1, pk1, sg1, 2, 1)

            @pl.when(2 * ds_ + 2 < NGRP)
            def _():
                unpack_group(2 * ds_ + 2, pk0, sg0, is0)
            return 0
        lax.fori_loop(1, NGRP // 2, dsuper, 0)

        wait_gather(pk1, GRP - 1, 1)
        scatter(sg1, GRP - 1, 1)
        wait_scatter(sg1, GRP - 2, 0)
        wait_scatter(sg1, GRP - 1, 1)
        plsc.subcore_barrier()

        lax.fori_loop(0, BLK, zfill, 0)
        fsem = (ss0, ss1)

        def flush_slab(j):
            pltpu.async_copy(acc.at[pl.ds(zbase + j * BLK, BLK)],
                             out.at[c, pl.ds(zbase + j * BLK, BLK)],
                             fsem[j % 2])

        def wait_flush_slab(j):
            pltpu.make_async_copy(acc.at[pl.ds(zbase + j * BLK, BLK)],
                                  out.at[c, pl.ds(zbase + j * BLK, BLK)],
                                  fsem[j % 2]).wait()

        flush_slab(0)
        for j in range(1, NSLAB):
            flush_slab(j)
            wait_flush_slab(j - 1)
            zero_slab(j - 1)
        pltpu.async_copy(acc.at[pl.ds(zbase + NSLAB * BLK, ZREM)],
                         out.at[c, pl.ds(zbase + NSLAB * BLK, ZREM)],
                         fsem[NSLAB % 2])
        wait_flush_slab(NSLAB - 1)
        zero_slab(NSLAB - 1)
        pltpu.make_async_copy(acc.at[pl.ds(zbase + NSLAB * BLK, ZREM)],
                              out.at[c, pl.ds(zbase + NSLAB * BLK, ZREM)],
                              fsem[NSLAB % 2]).wait()
        zero_rem()
        for z in range(NSLAB):
            wait_zero_slab(z)
        wait_zero_rem()
        plsc.subcore_barrier()
        return 0
    lax.fori_loop(0, CPC, chunk_body, 0)


def _sc_aggregate(h2, pk2, C):
    mesh = plsc.VectorSubcoreMesh(core_axis_name="c", subcore_axis_name="s")
    return pl.kernel(
        functools.partial(_sc_aggregate_body, C),
        out_type=jax.ShapeDtypeStruct((C, ACC_ROWS, W), jnp.float32),
        mesh=mesh,
        scratch_types=[
            pltpu.VMEM_SHARED((ACC_ROWS, W), jnp.float32),
            pltpu.VMEM((GRP, 128), jnp.int32),
            pltpu.VMEM((GRP, 128), jnp.int32),
            pltpu.VMEM((GRP, 128), jnp.int32),
            pltpu.VMEM((GRP, 128), jnp.int32),
            pltpu.VMEM((BLK, W), jnp.float32),
            pltpu.VMEM((BLK, W), jnp.float32),
            pltpu.SemaphoreType.DMA,
            pltpu.SemaphoreType.DMA,
            pltpu.SemaphoreType.DMA,
            pltpu.SemaphoreType.DMA,
            pltpu.SemaphoreType.DMA,
            pltpu.SemaphoreType.DMA,
        ],
        compiler_params=pltpu.CompilerParams(use_tc_tiling_on_sc=False),
        name="sc_rgcn_aggregate",
    )(h2, pk2)


def _sc_counts_body(pk2, out, acc, pkb, sgb, buf):
    core = lax.axis_index("c")
    sub = lax.axis_index("s")

    @pl.when(core == 0)
    def _():
        def fill(val):
            def f(i, _):
                for j in range(W // LANES):
                    buf[i, pl.ds(j * LANES, LANES)] = jnp.full(
                        (LANES,), val, jnp.float32)
                return 0
            lax.fori_loop(0, BLK, f, 0)

        fill(0.0)
        zbase = sub * TROWS
        for z in range(TROWS // BLK):
            pltpu.sync_copy(buf, acc.at[pl.ds(zbase + z * BLK, BLK)])
        rem = TROWS % BLK
        if rem:
            pltpu.sync_copy(buf.at[pl.ds(0, rem)],
                            acc.at[pl.ds(zbase + (TROWS // BLK) * BLK, rem)])
        fill(1.0)
        plsc.subcore_barrier()

        def grp_body(g, _):
            pltpu.sync_copy(pk2.at[pl.ds(sub * NBLKS + g * GRP, GRP)], pkb)
            for i in range(GRP):
                for j in range(128 // LANES):
                    v = pkb[i, pl.ds(j * LANES, LANES)]
                    sgb[i, pl.ds(j * LANES, LANES)] = v & SEGMASK
            for i in range(GRP):
                pltpu.sync_copy(buf, acc.at[sgb.at[i]], add=True)
            return 0
        lax.fori_loop(0, NGRP, grp_body, 0)
        plsc.subcore_barrier()

        fb = sub * TROWS
        pltpu.sync_copy(acc.at[pl.ds(fb, TROWS)], out.at[pl.ds(fb, TROWS)])

    return


def _sc_counts(pk2):
    mesh = plsc.VectorSubcoreMesh(core_axis_name="c", subcore_axis_name="s")
    return pl.kernel(
        _sc_counts_body,
        out_type=jax.ShapeDtypeStruct((ACC_ROWS, W), jnp.float32),
        mesh=mesh,
        scratch_types=[
            pltpu.VMEM_SHARED((ACC_ROWS, W), jnp.float32),
            pltpu.VMEM((GRP, 128), jnp.int32),
            pltpu.VMEM((GRP, 128), jnp.int32),
            pltpu.VMEM((BLK, W), jnp.float32),
        ],
        compiler_params=pltpu.CompilerParams(use_tc_tiling_on_sc=False),
        name="sc_rgcn_counts",
    )(pk2)


def _tc_wrel_body(basis_ref, comp_ref, out_ref):
    r = pl.program_id(1)
    acc = comp_ref[r, 0] * basis_ref[0]
    acc = acc + comp_ref[r, 1] * basis_ref[1]
    acc = acc + comp_ref[r, 2] * basis_ref[2]
    out_ref[0] = acc


def _tc_wrel(basis, comp, C, dout):
    return pl.pallas_call(
        _tc_wrel_body,
        grid=(C, R),
        in_specs=[
            pl.BlockSpec((3, W, dout), lambda c, r: (0, c, 0)),
            pl.BlockSpec(memory_space=pltpu.SMEM),
        ],
        out_specs=pl.BlockSpec((1, W, dout), lambda c, r: (c, r, 0)),
        out_shape=jax.ShapeDtypeStruct((C, R * W, dout), jnp.float32),
        name="tc_wrel",
    )(basis, comp)


def _tc_inv_body(cnt_ref, out_ref):
    out_ref[...] = 1.0 / jnp.maximum(cnt_ref[...], 1.0)


def _tc_inv(cntx):
    return pl.pallas_call(
        _tc_inv_body,
        grid=(8,),
        in_specs=[pl.BlockSpec((NPAD // 8, R * W), lambda m: (m, 0))],
        out_specs=pl.BlockSpec((NPAD // 8, R * W), lambda m: (m, 0)),
        out_shape=jax.ShapeDtypeStruct((NPAD, R * W), jnp.float32),
        name="tc_inv_counts",
    )(cntx)


def _tc_layer_body(C2, act, h_ref, root_ref, wrel_ref, sums_ref, inv_ref,
                   bias_ref, out_ref, acc_ref):
    m = pl.program_id(0)
    k = pl.program_id(1)
    invb = inv_ref[...]
    part = (jnp.dot(sums_ref[0] * invb, wrel_ref[0],
                    preferred_element_type=jnp.float32)
            + jnp.dot(sums_ref[1] * invb, wrel_ref[1],
                      preferred_element_type=jnp.float32))

    @pl.when(k == 0)
    def _():
        acc_ref[...] = part + jnp.dot(h_ref[...], root_ref[...],
                                      preferred_element_type=jnp.float32)

    @pl.when(k > 0)
    def _():
        acc_ref[...] = acc_ref[...] + part

    @pl.when(k == C2 - 1)
    def _():
        val = acc_ref[...] + bias_ref[...]
        if act == "relu":
            out_ref[...] = jnp.maximum(val, 0.0)
        else:
            val = jnp.tanh(val)
            bmax = jnp.max(val, axis=0, keepdims=True)

            @pl.when(m == 0)
            def _():
                out_ref[...] = bmax

            @pl.when(m > 0)
            def _():
                out_ref[...] = jnp.maximum(out_ref[...], bmax)


def _tc_layer(h, root, wrel, sums, inv, bias, act):
    din, dout = root.shape
    C2 = din // W // 2
    if act == "relu":
        out_shape = jax.ShapeDtypeStruct((N, dout), jnp.float32)
        out_spec = pl.BlockSpec((BM, dout), lambda m, k: (m, 0))
    else:
        out_shape = jax.ShapeDtypeStruct((1, dout), jnp.float32)
        out_spec = pl.BlockSpec((1, dout), lambda m, k: (0, 0))
    return pl.pallas_call(
        functools.partial(_tc_layer_body, C2, act),
        grid=(N // BM, C2),
        in_specs=[
            pl.BlockSpec((BM, din), lambda m, k: (m, 0)),
            pl.BlockSpec((din, dout), lambda m, k: (0, 0)),
            pl.BlockSpec((2, R * W, dout), lambda m, k: (k, 0, 0)),
            pl.BlockSpec((2, BM, R * W), lambda m, k: (k, m, 0)),
            pl.BlockSpec((BM, R * W), lambda m, k: (m, 0)),
            pl.BlockSpec((1, dout), lambda m, k: (0, 0)),
        ],
        out_specs=out_spec,
        out_shape=out_shape,
        scratch_shapes=[pltpu.VMEM((BM, dout), jnp.float32)],
        name="tc_rgcn_layer",
    )(h, root, wrel, sums, inv, bias)


def kernel(x, edge_index, edge_type, basis0, comp0, root0, bias0,
           basis1, comp1, root1, bias1, basis2, comp2, root2, bias2):
    src = edge_index[0]
    dst = edge_index[1]
    seg = dst * R + edge_type
    pk = jnp.bitwise_or(jnp.left_shift(src, 17), seg)
    pad = EPAD - E
    pk2 = jnp.concatenate(
        [pk, jnp.full((pad,), NSEG, jnp.int32)]).reshape(EPAD // 128, 128)

    cntx = _sc_counts(pk2).reshape(NPAD, R * W)
    inv = _tc_inv(cntx)

    h = x
    layers = [(basis0, comp0, root0, bias0, "relu"),
              (basis1, comp1, root1, bias1, "relu"),
              (basis2, comp2, root2, bias2, "tanh")]
    for basis, comp, root, bias, act in layers:
        din, dout = root.shape
        C = din // W
        wrel = _tc_wrel(basis, comp, C, dout)
        sums = _sc_aggregate(h.reshape(N * C, W), pk2, C)
        h = _tc_layer(h, root, wrel, sums.reshape(C, NPAD, R * W), inv,
                      bias.reshape(1, dout), act)
    return h
